# Initial kernel scaffold; baseline (speedup 1.0000x reference)
import jax
import jax.numpy as jnp
from jax.experimental import pallas as pl


def kernel(x, targets, negatives, embeddings, out_weight):
    raise NotImplementedError("write your pallas kernel here")


# speedup vs baseline: 38.7805x; 38.7805x over previous
import functools

import jax
import jax.numpy as jnp
from jax import lax
from jax.experimental import pallas as pl
from jax.experimental.pallas import tpu as pltpu
from jax.experimental.pallas import tpu_sc as plsc

VOCAB = 1000
DIM = 128
B = 16384
NEG = 50
VP = 1024
G = NEG + 1
NC, NS = 2, 16
NW = NC * NS
SPW = B // NW
CPG = SPW // 128
NCHUNK = G * CPG


def _matmul_body(e_ref, w_ref, s_ref):
    s_ref[...] = lax.dot_general(
        e_ref[...], w_ref[...],
        (((1,), (1,)), ((), ())),
        preferred_element_type=jnp.float32,
    )


def _scores(emb_pad, w_pad):
    return pl.pallas_call(
        _matmul_body,
        out_shape=jax.ShapeDtypeStruct((VP, VP), jnp.float32),
    )(emb_pad, w_pad)


def _sc_gather_body(s_hbm, x_hbm, cols_hbm, out_hbm, xs_v, tv_v, idx_v,
                    vals_v, sem):
    wid = lax.axis_index("s") * NC + lax.axis_index("c")
    base = wid * SPW
    pltpu.sync_copy(x_hbm.at[pl.ds(base, SPW)], xs_v)
    for i in range(SPW // 16):
        xs_v[pl.ds(i * 16, 16)] = xs_v[pl.ds(i * 16, 16)] * VP
    pltpu.sync_copy(cols_hbm.at[wid], tv_v)

    def chunk_body(c, carry):
        q = (c % CPG) * 128
        p = c * 128
        for u in range(8):
            idx_v[pl.ds(p + u * 16, 16)] = (
                xs_v[pl.ds(q + u * 16, 16)] + tv_v[pl.ds(p + u * 16, 16)]
            )
        return carry

    lax.fori_loop(0, NCHUNK, chunk_body, 0)

    pltpu.async_copy(s_hbm.at[idx_v], vals_v, sem).wait()
    pltpu.sync_copy(vals_v, out_hbm.at[wid])


@functools.cache
def _sc_gather():
    return pl.kernel(
        _sc_gather_body,
        out_type=jax.ShapeDtypeStruct((NW, NCHUNK * 128), jnp.float32),
        mesh=plsc.VectorSubcoreMesh(core_axis_name="c", subcore_axis_name="s"),
        scratch_types=[
            pltpu.VMEM((SPW,), jnp.int32),
            pltpu.VMEM((NCHUNK * 128,), jnp.int32),
            pltpu.VMEM((NCHUNK * 128,), jnp.int32),
            pltpu.VMEM((NCHUNK * 128,), jnp.float32),
            pltpu.SemaphoreType.DMA,
        ],
    )


def _loss_body(v_ref, o_ref):
    v = v_ref[...]
    r = lax.broadcasted_iota(jnp.int32, v.shape, 0)
    is_pos = (r % NCHUNK) < CPG
    z = jnp.where(is_pos, -v, v)
    sp = jnp.maximum(z, 0.0) + jnp.log1p(jnp.exp(-jnp.abs(z)))
    o_ref[0, 0] = jnp.sum(sp) * (1.0 / B)


def _loss(vals2d):
    out = pl.pallas_call(
        _loss_body,
        out_shape=jax.ShapeDtypeStruct((1, 1), jnp.float32),
        out_specs=pl.BlockSpec(memory_space=pltpu.SMEM),
    )(vals2d)
    return out[0, 0]


def kernel(x, targets, negatives, embeddings, out_weight):
    x = x.astype(jnp.int32)
    targets = targets.astype(jnp.int32)
    negatives = negatives.astype(jnp.int32)
    emb_pad = jnp.zeros((VP, DIM), jnp.float32).at[:VOCAB].set(embeddings)
    w_pad = jnp.zeros((VP, DIM), jnp.float32).at[:VOCAB].set(out_weight)

    s = _scores(emb_pad, w_pad).reshape(VP * VP)

    cols = jnp.concatenate([targets[None, :], negatives.T], axis=0)
    cols = cols.reshape(G, NW, SPW).transpose(1, 0, 2).reshape(NW, NCHUNK * 128)

    vals = _sc_gather()(s, x, cols)
    return _loss(vals.reshape(NW * NCHUNK, 128))


# speedup vs baseline: 42.2079x; 1.0884x over previous
import functools

import jax
import jax.numpy as jnp
from jax import lax
from jax.experimental import pallas as pl
from jax.experimental.pallas import tpu as pltpu
from jax.experimental.pallas import tpu_sc as plsc

VOCAB = 1000
DIM = 128
B = 16384
NEG = 50
VP = 1024
CB = VP // DIM
VH = 504
G = NEG + 1
NC, NS = 2, 16
NW = NC * NS
SPW = B // NW
CPG = SPW // 128
NCHUNK = G * CPG
SFLAT = CB * VH * DIM
CWSTR = VH * DIM - DIM
SSLICE = SFLAT // NS
HI16 = -65536


def _matmul_body(ee_ref, eo_ref, w_ref, s_ref):
    a = lax.dot_general(ee_ref[...], w_ref[...], (((1,), (1,)), ((), ())),
                        preferred_element_type=jnp.float32)
    b = lax.dot_general(eo_ref[...], w_ref[...], (((1,), (1,)), ((), ())),
                        preferred_element_type=jnp.float32)
    a16 = lax.bitcast_convert_type(a.astype(jnp.bfloat16), jnp.uint16)
    b16 = lax.bitcast_convert_type(b.astype(jnp.bfloat16), jnp.uint16)
    word = a16.astype(jnp.uint32) | (b16.astype(jnp.uint32) << 16)
    s_ref[0, :, :] = word.astype(jnp.int32)


def _scores(emb_even, emb_odd, w_pad):
    return pl.pallas_call(
        _matmul_body,
        grid=(CB,),
        in_specs=[
            pl.BlockSpec((VH, DIM), lambda cb: (0, 0)),
            pl.BlockSpec((VH, DIM), lambda cb: (0, 0)),
            pl.BlockSpec((DIM, DIM), lambda cb: (cb, 0)),
        ],
        out_specs=pl.BlockSpec((1, VH, DIM), lambda cb: (cb, 0, 0)),
        out_shape=jax.ShapeDtypeStruct((CB, VH, DIM), jnp.int32),
    )(emb_even, emb_odd, w_pad)


def _sc_gather_body(s_hbm, x_hbm, t_hbm, neg_hbm, out_hbm, xs_v, par_v,
                    negv_v, idx_v, wv_v, s_shared, sem, sem2):
    cid = lax.axis_index("c")
    sid = lax.axis_index("s")
    wid = sid * NC + cid
    base = wid * SPW

    stage = pltpu.make_async_copy(
        s_hbm.at[pl.ds(sid * SSLICE, SSLICE)],
        s_shared.at[pl.ds(sid * SSLICE, SSLICE)],
        sem2,
    )
    stage.start()

    pltpu.sync_copy(x_hbm.at[pl.ds(base, SPW)], xs_v)
    for i in range(SPW // 16):
        xv = xs_v[pl.ds(i * 16, 16)]
        par_v[pl.ds(i * 16, 16)] = xv & 1
        xs_v[pl.ds(i * 16, 16)] = (xv >> 1) * 128

    pltpu.sync_copy(t_hbm.at[pl.ds(base, SPW)], negv_v.at[pl.ds(0, SPW)])
    for i in range(SPW // 16):
        c = negv_v[pl.ds(i * 16, 16)]
        idx_v[pl.ds(i * 16, 16)] = (
            xs_v[pl.ds(i * 16, 16)] + c + (c >> 7) * CWSTR
        )

    pltpu.sync_copy(neg_hbm.at[pl.ds(base * NEG, SPW * NEG)], negv_v)
    iota_neg = lax.iota(jnp.int32, 16) * NEG

    def chunk_body(cc, carry):
        j = cc // CPG - 1
        k = cc % CPG
        for u in range(8):
            s0 = k * 128 + u * 16
            gidx = iota_neg + (s0 * NEG + j)
            c = plsc.load_gather(negv_v, [gidx])
            idx_v[pl.ds(cc * 128 + u * 16, 16)] = (
                xs_v[pl.ds(s0, 16)] + c + (c >> 7) * CWSTR
            )
        return carry

    lax.fori_loop(CPG, NCHUNK, chunk_body, 0)

    stage.wait()
    plsc.subcore_barrier()
    pltpu.async_copy(s_shared.at[idx_v], wv_v, sem).wait()

    def unpack_body(cc, carry):
        k = cc % CPG
        for u in range(8):
            q = cc * 128 + u * 16
            w = wv_v[pl.ds(q, 16)]
            p = par_v[pl.ds(k * 128 + u * 16, 16)]
            wv_v[pl.ds(q, 16)] = jnp.where(p == 1, w & HI16, w << 16)
        return carry

    lax.fori_loop(0, NCHUNK, unpack_body, 0)

    po = pl.multiple_of(wid * SPW, SPW)
    pltpu.sync_copy(wv_v.at[pl.ds(0, SPW)], out_hbm.at[pl.ds(po, SPW)])
    no = pl.multiple_of(B + wid * (SPW * NEG), SPW)
    pltpu.sync_copy(wv_v.at[pl.ds(SPW, SPW * NEG)],
                    out_hbm.at[pl.ds(no, SPW * NEG)])


@functools.cache
def _sc_gather():
    return pl.kernel(
        _sc_gather_body,
        out_type=jax.ShapeDtypeStruct((B * G,), jnp.int32),
        mesh=plsc.VectorSubcoreMesh(core_axis_name="c", subcore_axis_name="s"),
        compiler_params=pltpu.CompilerParams(needs_layout_passes=False),
        scratch_types=[
            pltpu.VMEM((SPW,), jnp.int32),
            pltpu.VMEM((SPW,), jnp.int32),
            pltpu.VMEM((SPW * NEG,), jnp.int32),
            pltpu.VMEM((NCHUNK * 128,), jnp.int32),
            pltpu.VMEM((NCHUNK * 128,), jnp.int32),
            pltpu.VMEM_SHARED((SFLAT,), jnp.int32),
            pltpu.SemaphoreType.DMA,
            pltpu.SemaphoreType.DMA,
        ],
    )


def _loss_body(v_ref, o_ref):
    bits = v_ref[...].reshape(B * G // 128, 128)
    v = lax.bitcast_convert_type(bits, jnp.float32)
    r = lax.broadcasted_iota(jnp.int32, v.shape, 0)
    is_pos = r < B // 128
    z = jnp.where(is_pos, -v, v)
    sp = jnp.maximum(z, 0.0) + jnp.log1p(jnp.exp(-jnp.abs(z)))
    o_ref[0, 0] = jnp.sum(sp) * (1.0 / B)


def _loss(vals):
    out = pl.pallas_call(
        _loss_body,
        out_shape=jax.ShapeDtypeStruct((1, 1), jnp.float32),
        out_specs=pl.BlockSpec(memory_space=pltpu.SMEM),
    )(vals)
    return out[0, 0]


def kernel(x, targets, negatives, embeddings, out_weight):
    x = x.astype(jnp.int32)
    targets = targets.astype(jnp.int32)
    negatives = negatives.astype(jnp.int32)
    w_pad = jnp.zeros((VP, DIM), jnp.float32).at[:VOCAB].set(out_weight)
    emb_even = jnp.zeros((VH, DIM), jnp.float32).at[:VOCAB // 2].set(
        embeddings[0::2])
    emb_odd = jnp.zeros((VH, DIM), jnp.float32).at[:VOCAB // 2].set(
        embeddings[1::2])

    s = _scores(emb_even, emb_odd, w_pad).reshape(SFLAT)
    vals = _sc_gather()(s, x, targets, negatives.reshape(B * NEG))
    return _loss(vals)


# speedup vs baseline: 52.0550x; 1.2333x over previous
import functools

import jax
import jax.numpy as jnp
from jax import lax
from jax.experimental import pallas as pl
from jax.experimental.pallas import tpu as pltpu
from jax.experimental.pallas import tpu_sc as plsc

VOCAB = 1000
DIM = 128
B = 16384
NEG = 50
VP = 1024
CB = VP // DIM
VH = 504
G = NEG + 1
NC, NS = 2, 16
NW = NC * NS
SPW = B // NW
CPG = SPW // 128
NCHUNK = G * CPG
MID = 104
SFLAT = CB * VH * DIM
CWSTR = VH * DIM - DIM
SSLICE = SFLAT // NS
HI16 = -65536


def _matmul_body(ee_ref, eo_ref, w_ref, s_ref):
    a = lax.dot_general(ee_ref[...], w_ref[...], (((1,), (1,)), ((), ())),
                        preferred_element_type=jnp.float32)
    b = lax.dot_general(eo_ref[...], w_ref[...], (((1,), (1,)), ((), ())),
                        preferred_element_type=jnp.float32)
    a16 = lax.bitcast_convert_type(a.astype(jnp.bfloat16), jnp.uint16)
    b16 = lax.bitcast_convert_type(b.astype(jnp.bfloat16), jnp.uint16)
    word = a16.astype(jnp.uint32) | (b16.astype(jnp.uint32) << 16)
    s_ref[0, :, :] = word.astype(jnp.int32)


def _scores(emb_even, emb_odd, w_pad):
    return pl.pallas_call(
        _matmul_body,
        grid=(CB,),
        in_specs=[
            pl.BlockSpec((VH, DIM), lambda cb: (0, 0)),
            pl.BlockSpec((VH, DIM), lambda cb: (0, 0)),
            pl.BlockSpec((DIM, DIM), lambda cb: (cb, 0)),
        ],
        out_specs=pl.BlockSpec((1, VH, DIM), lambda cb: (cb, 0, 0)),
        out_shape=jax.ShapeDtypeStruct((CB, VH, DIM), jnp.int32),
    )(emb_even, emb_odd, w_pad)


def _sc_gather_body(s_hbm, x_hbm, t_hbm, neg_hbm, out_hbm, xs_v, shv_v,
                    negv_v, idx_v, wv_v, s_shared, sem_a, sem_b, sem_s, sem_w):
    cid = lax.axis_index("c")
    sid = lax.axis_index("s")
    wid = sid * NC + cid
    base = wid * SPW

    stage = pltpu.make_async_copy(
        s_hbm.at[pl.ds(sid * SSLICE, SSLICE)],
        s_shared.at[pl.ds(sid * SSLICE, SSLICE)],
        sem_s,
    )
    stage.start()

    pltpu.sync_copy(x_hbm.at[pl.ds(base, SPW)], xs_v)
    for i in range(SPW // 16):
        xv = xs_v[pl.ds(i * 16, 16)]
        shv_v[pl.ds(i * 16, 16)] = (xv & 1) << 4
        xs_v[pl.ds(i * 16, 16)] = (xv >> 1) * 128

    pltpu.sync_copy(t_hbm.at[pl.ds(base, SPW)], negv_v.at[pl.ds(0, SPW)])
    for i in range(SPW // 16):
        c = negv_v[pl.ds(i * 16, 16)]
        idx_v[pl.ds(i * 16, 16)] = (
            xs_v[pl.ds(i * 16, 16)] + c + (c >> 7) * CWSTR
        )

    pltpu.sync_copy(neg_hbm.at[pl.ds(base * NEG, SPW * NEG)], negv_v)
    iota_neg = lax.iota(jnp.int32, 16) * NEG

    def build(cc):
        j = cc // CPG - 1
        k = cc % CPG
        for u in range(8):
            s0 = k * 128 + u * 16
            gidx = iota_neg + (s0 * NEG + j)
            c = plsc.load_gather(negv_v, [gidx])
            idx_v[pl.ds(cc * 128 + u * 16, 16)] = (
                xs_v[pl.ds(s0, 16)] + c + (c >> 7) * CWSTR
            )

    def unpack(cc):
        k = cc % CPG
        for u in range(8):
            q = cc * 128 + u * 16
            w = wv_v[pl.ds(q, 16)]
            sh = shv_v[pl.ds(k * 128 + u * 16, 16)]
            wv_v[pl.ds(q, 16)] = (w >> sh) << 16

    plsc.parallel_loop(CPG, MID)(build)
    stage.wait()
    plsc.subcore_barrier()
    ga = pltpu.make_async_copy(
        s_shared.at[idx_v.at[pl.ds(0, MID * 128)]],
        wv_v.at[pl.ds(0, MID * 128)], sem_a)
    ga.start()
    plsc.parallel_loop(MID, NCHUNK)(build)
    gb = pltpu.make_async_copy(
        s_shared.at[idx_v.at[pl.ds(MID * 128, (NCHUNK - MID) * 128)]],
        wv_v.at[pl.ds(MID * 128, (NCHUNK - MID) * 128)], sem_b)
    gb.start()
    ga.wait()
    plsc.parallel_loop(0, MID)(unpack)

    po = pl.multiple_of(wid * SPW, SPW)
    w1 = pltpu.make_async_copy(wv_v.at[pl.ds(0, SPW)],
                               out_hbm.at[pl.ds(po, SPW)], sem_w)
    w1.start()
    n1 = pl.multiple_of(B + wid * (SPW * NEG), SPW)
    w2 = pltpu.make_async_copy(
        wv_v.at[pl.ds(SPW, (MID - CPG) * 128)],
        out_hbm.at[pl.ds(n1, (MID - CPG) * 128)], sem_w)
    w2.start()
    gb.wait()
    plsc.parallel_loop(MID, NCHUNK)(unpack)
    n2 = pl.multiple_of(B + wid * (SPW * NEG) + (MID - CPG) * 128, SPW)
    w3 = pltpu.make_async_copy(
        wv_v.at[pl.ds(MID * 128, (NCHUNK - MID) * 128)],
        out_hbm.at[pl.ds(n2, (NCHUNK - MID) * 128)], sem_w)
    w3.start()
    w1.wait()
    w2.wait()
    w3.wait()


@functools.cache
def _sc_gather():
    return pl.kernel(
        _sc_gather_body,
        out_type=jax.ShapeDtypeStruct((B * G,), jnp.int32),
        mesh=plsc.VectorSubcoreMesh(core_axis_name="c", subcore_axis_name="s"),
        compiler_params=pltpu.CompilerParams(needs_layout_passes=False),
        scratch_types=[
            pltpu.VMEM((SPW,), jnp.int32),
            pltpu.VMEM((SPW,), jnp.int32),
            pltpu.VMEM((SPW * NEG,), jnp.int32),
            pltpu.VMEM((NCHUNK * 128,), jnp.int32),
            pltpu.VMEM((NCHUNK * 128,), jnp.int32),
            pltpu.VMEM_SHARED((SFLAT,), jnp.int32),
            pltpu.SemaphoreType.DMA,
            pltpu.SemaphoreType.DMA,
            pltpu.SemaphoreType.DMA,
            pltpu.SemaphoreType.DMA,
        ],
    )


def _loss_body(v_ref, o_ref):
    bits = v_ref[...].reshape(B * G // 128, 128)
    v = lax.bitcast_convert_type(bits, jnp.float32)
    r = lax.broadcasted_iota(jnp.int32, v.shape, 0)
    is_pos = r < B // 128
    z = jnp.where(is_pos, -v, v)
    sp = jnp.maximum(z, 0.0) + jnp.log1p(jnp.exp(-jnp.abs(z)))
    o_ref[0, 0] = jnp.sum(sp) * (1.0 / B)


def _loss(vals):
    out = pl.pallas_call(
        _loss_body,
        out_shape=jax.ShapeDtypeStruct((1, 1), jnp.float32),
        out_specs=pl.BlockSpec(memory_space=pltpu.SMEM),
    )(vals)
    return out[0, 0]


def kernel(x, targets, negatives, embeddings, out_weight):
    x = x.astype(jnp.int32)
    targets = targets.astype(jnp.int32)
    negatives = negatives.astype(jnp.int32)
    w_pad = jnp.zeros((VP, DIM), jnp.float32).at[:VOCAB].set(out_weight)
    emb_even = jnp.zeros((VH, DIM), jnp.float32).at[:VOCAB // 2].set(
        embeddings[0::2])
    emb_odd = jnp.zeros((VH, DIM), jnp.float32).at[:VOCAB // 2].set(
        embeddings[1::2])

    s = _scores(emb_even, emb_odd, w_pad).reshape(SFLAT)
    vals = _sc_gather()(s, x, targets, negatives.reshape(B * NEG))
    return _loss(vals)


# speedup vs baseline: 55.7841x; 1.0716x over previous
import functools

import jax
import jax.numpy as jnp
from jax import lax
from jax.experimental import pallas as pl
from jax.experimental.pallas import tpu as pltpu
from jax.experimental.pallas import tpu_sc as plsc

VOCAB = 1000
DIM = 128
B = 16384
NEG = 50
VP = 1024
CB = VP // DIM
VH = 504
G = NEG + 1
NC, NS = 2, 16
NW = NC * NS
SPW = B // NW
CPG = SPW // 128
NCHUNK = G * CPG
MID = 104
SFLAT = CB * VH * DIM
CWSTR = VH * DIM - DIM
SSLICE = SFLAT // NS
HI16 = -65536


def _matmul_body(e_ref, w_ref, s_ref):
    e = e_ref[...]
    ee = e[:VH, :]
    eo = e[VH:, :]
    a = lax.dot_general(ee, w_ref[...], (((1,), (1,)), ((), ())),
                        preferred_element_type=jnp.float32)
    b = lax.dot_general(eo, w_ref[...], (((1,), (1,)), ((), ())),
                        preferred_element_type=jnp.float32)
    a16 = lax.bitcast_convert_type(a.astype(jnp.bfloat16), jnp.uint16)
    b16 = lax.bitcast_convert_type(b.astype(jnp.bfloat16), jnp.uint16)
    word = a16.astype(jnp.uint32) | (b16.astype(jnp.uint32) << 16)
    s_ref[...] = word.astype(jnp.int32).reshape(VH * DIM)


def _scores(emb_pad, w_pad):
    return pl.pallas_call(
        _matmul_body,
        grid=(CB,),
        in_specs=[
            pl.BlockSpec((2 * VH, DIM), lambda cb: (0, 0)),
            pl.BlockSpec((DIM, DIM), lambda cb: (cb, 0)),
        ],
        out_specs=pl.BlockSpec((VH * DIM,), lambda cb: (cb,)),
        out_shape=jax.ShapeDtypeStruct((SFLAT,), jnp.int32),
    )(emb_pad, w_pad)


def _sc_gather_body(s_hbm, x_hbm, t_hbm, neg_hbm, out_hbm, xs_v, shv_v,
                    negv_v, idx_v, wv_v, s_shared, sem_a, sem_b, sem_s, sem_w):
    cid = lax.axis_index("c")
    sid = lax.axis_index("s")
    wid = sid * NC + cid
    base = wid * SPW

    stage = pltpu.make_async_copy(
        s_hbm.at[pl.ds(sid * SSLICE, SSLICE)],
        s_shared.at[pl.ds(sid * SSLICE, SSLICE)],
        sem_s,
    )
    stage.start()

    pltpu.sync_copy(x_hbm.at[pl.ds(base, SPW)], xs_v)
    for i in range(SPW // 16):
        xv = xs_v[pl.ds(i * 16, 16)]
        hi = jnp.where(xv >= VH, 1, 0)
        shv_v[pl.ds(i * 16, 16)] = hi << 4
        xs_v[pl.ds(i * 16, 16)] = (xv - hi * VH) * 128

    pltpu.sync_copy(t_hbm.at[pl.ds(base, SPW)], negv_v.at[pl.ds(0, SPW)])
    for i in range(SPW // 16):
        c = negv_v[pl.ds(i * 16, 16)]
        idx_v[pl.ds(i * 16, 16)] = (
            xs_v[pl.ds(i * 16, 16)] + c + (c >> 7) * CWSTR
        )

    pltpu.sync_copy(neg_hbm.at[pl.ds(base * NEG, SPW * NEG)], negv_v)
    iota_neg = lax.iota(jnp.int32, 16) * NEG

    def build(cc):
        j = cc // CPG - 1
        k = cc % CPG
        for u in range(8):
            s0 = k * 128 + u * 16
            gidx = iota_neg + (s0 * NEG + j)
            c = plsc.load_gather(negv_v, [gidx])
            idx_v[pl.ds(cc * 128 + u * 16, 16)] = (
                xs_v[pl.ds(s0, 16)] + c + (c >> 7) * CWSTR
            )

    def unpack(cc):
        k = cc % CPG
        for u in range(8):
            q = cc * 128 + u * 16
            w = wv_v[pl.ds(q, 16)]
            sh = shv_v[pl.ds(k * 128 + u * 16, 16)]
            wv_v[pl.ds(q, 16)] = (w >> sh) << 16

    plsc.parallel_loop(CPG, MID)(build)
    stage.wait()
    plsc.subcore_barrier()
    ga = pltpu.make_async_copy(
        s_shared.at[idx_v.at[pl.ds(0, MID * 128)]],
        wv_v.at[pl.ds(0, MID * 128)], sem_a)
    ga.start()
    plsc.parallel_loop(MID, NCHUNK)(build)
    gb = pltpu.make_async_copy(
        s_shared.at[idx_v.at[pl.ds(MID * 128, (NCHUNK - MID) * 128)]],
        wv_v.at[pl.ds(MID * 128, (NCHUNK - MID) * 128)], sem_b)
    gb.start()
    ga.wait()
    plsc.parallel_loop(0, MID)(unpack)

    po = pl.multiple_of(wid * SPW, SPW)
    w1 = pltpu.make_async_copy(wv_v.at[pl.ds(0, SPW)],
                               out_hbm.at[pl.ds(po, SPW)], sem_w)
    w1.start()
    n1 = pl.multiple_of(B + wid * (SPW * NEG), SPW)
    w2 = pltpu.make_async_copy(
        wv_v.at[pl.ds(SPW, (MID - CPG) * 128)],
        out_hbm.at[pl.ds(n1, (MID - CPG) * 128)], sem_w)
    w2.start()
    gb.wait()
    plsc.parallel_loop(MID, NCHUNK)(unpack)
    n2 = pl.multiple_of(B + wid * (SPW * NEG) + (MID - CPG) * 128, SPW)
    w3 = pltpu.make_async_copy(
        wv_v.at[pl.ds(MID * 128, (NCHUNK - MID) * 128)],
        out_hbm.at[pl.ds(n2, (NCHUNK - MID) * 128)], sem_w)
    w3.start()
    w1.wait()
    w2.wait()
    w3.wait()


@functools.cache
def _sc_gather():
    return pl.kernel(
        _sc_gather_body,
        out_type=jax.ShapeDtypeStruct((B * G,), jnp.int32),
        mesh=plsc.VectorSubcoreMesh(core_axis_name="c", subcore_axis_name="s"),
        compiler_params=pltpu.CompilerParams(needs_layout_passes=False),
        scratch_types=[
            pltpu.VMEM((SPW,), jnp.int32),
            pltpu.VMEM((SPW,), jnp.int32),
            pltpu.VMEM((SPW * NEG,), jnp.int32),
            pltpu.VMEM((NCHUNK * 128,), jnp.int32),
            pltpu.VMEM((NCHUNK * 128,), jnp.int32),
            pltpu.VMEM_SHARED((SFLAT,), jnp.int32),
            pltpu.SemaphoreType.DMA,
            pltpu.SemaphoreType.DMA,
            pltpu.SemaphoreType.DMA,
            pltpu.SemaphoreType.DMA,
        ],
    )


def _loss_body(v_ref, o_ref):
    bits = v_ref[...].reshape(B * G // 128, 128)
    v = lax.bitcast_convert_type(bits, jnp.float32)
    r = lax.broadcasted_iota(jnp.int32, v.shape, 0)
    is_pos = r < B // 128
    z = jnp.where(is_pos, -v, v)
    sp = jnp.maximum(z, 0.0) + jnp.log1p(jnp.exp(-jnp.abs(z)))
    o_ref[0, 0] = jnp.sum(sp) * (1.0 / B)


def _loss(vals):
    out = pl.pallas_call(
        _loss_body,
        out_shape=jax.ShapeDtypeStruct((1, 1), jnp.float32),
        out_specs=pl.BlockSpec(memory_space=pltpu.SMEM),
    )(vals)
    return out[0, 0]


def kernel(x, targets, negatives, embeddings, out_weight):
    x = x.astype(jnp.int32)
    targets = targets.astype(jnp.int32)
    negatives = negatives.astype(jnp.int32)
    w_pad = jnp.zeros((VP, DIM), jnp.float32).at[:VOCAB].set(out_weight)
    emb_pad = jnp.zeros((2 * VH, DIM), jnp.float32).at[:VOCAB].set(embeddings)

    s = _scores(emb_pad, w_pad)
    vals = _sc_gather()(s, x, targets, negatives.reshape(B * NEG))
    return _loss(vals)
